# baseline (speedup 1.0000x reference)
import dataclasses

import jax
import jax.numpy as jnp
from jax import lax
from jax.experimental import pallas as pl
from jax.experimental.pallas import tpu as pltpu
from jax.experimental.pallas import tpu_sc as plsc

SIZE = 256
B = 64
N = 16384
NUM_PIX = SIZE * SIZE
NC = 2
NS = 16
NW = NC * NS
BPW = B // NW
L = 16


def _splat_body(c01_hbm, val_hbm, out_hbm, acc_v, c0_v, c1_v, val_v, sem):
    wid = lax.axis_index("s") * NC + lax.axis_index("c")
    zeros = jnp.zeros((L,), jnp.float32)

    for r in range(BPW):
        b = wid * BPW + r

        cp0 = pltpu.async_copy(c01_hbm.at[0, b], c0_v, sem)
        cp1 = pltpu.async_copy(c01_hbm.at[1, b], c1_v, sem)
        cp2 = pltpu.async_copy(val_hbm.at[b], val_v, sem)

        @pl.loop(0, NUM_PIX, step=4 * L)
        def _(i):
            acc_v[pl.ds(i, L)] = zeros
            acc_v[pl.ds(i + L, L)] = zeros
            acc_v[pl.ds(i + 2 * L, L)] = zeros
            acc_v[pl.ds(i + 3 * L, L)] = zeros

        cp0.wait()
        cp1.wait()
        cp2.wait()

        def _points(i):
            c0 = c0_v[pl.ds(i, L)]
            c1 = c1_v[pl.ds(i, L)]
            v = val_v[pl.ds(i, L)]
            f0 = c0.astype(jnp.int32)
            f1 = c1.astype(jnp.int32)
            fr0 = c0 - f0.astype(jnp.float32)
            fr1 = c1 - f1.astype(jnp.float32)
            bv = fr0 * v
            av = v - bv
            p_fc = av * fr1
            p_ff = av - p_fc
            p_cc = bv * fr1
            p_cf = bv - p_cc
            i_ff = f0 * SIZE + f1
            plsc.addupdate_scatter(acc_v, [i_ff], p_ff)
            plsc.addupdate_scatter(acc_v, [i_ff + 1], p_fc)
            plsc.addupdate_scatter(acc_v, [i_ff + SIZE], p_cf)
            plsc.addupdate_scatter(acc_v, [i_ff + SIZE + 1], p_cc)

        @plsc.parallel_loop(0, N, step=L, unroll=2)
        def _(i):
            _points(i)

        pltpu.sync_copy(acc_v.at[pl.ds(0, NUM_PIX)], out_hbm.at[b])


def kernel(values, coord):
    one = values[0, 0] * 0.0 + 1.0
    c01 = jnp.transpose(coord, (2, 0, 1)) * one
    mesh = plsc.VectorSubcoreMesh(core_axis_name="c", subcore_axis_name="s")
    cp = pltpu.CompilerParams()
    if "needs_layout_passes" in pltpu.CompilerParams.__dataclass_fields__:
        cp = dataclasses.replace(cp, needs_layout_passes=False)
    splat = pl.kernel(
        _splat_body,
        out_type=jax.ShapeDtypeStruct((B, NUM_PIX), jnp.float32),
        mesh=mesh,
        scratch_types=[
            pltpu.VMEM((NUM_PIX + 2 * SIZE,), jnp.float32),
            pltpu.VMEM((N,), jnp.float32),
            pltpu.VMEM((N,), jnp.float32),
            pltpu.VMEM((N,), jnp.float32),
            pltpu.SemaphoreType.DMA,
        ],
        compiler_params=cp,
    )
    img = splat(c01, values)
    return img.reshape(B, 1, SIZE, SIZE)


# speedup vs baseline: 1.0179x; 1.0179x over previous
import dataclasses

import jax
import jax.numpy as jnp
from jax import lax
from jax.experimental import pallas as pl
from jax.experimental.pallas import tpu as pltpu
from jax.experimental.pallas import tpu_sc as plsc

SIZE = 256
B = 64
N = 16384
NUM_PIX = SIZE * SIZE
NC = 2
NS = 16
NW = NC * NS
BPW = B // NW
L = 16


def _splat_body(c01_hbm, val_hbm, out_hbm, acc_v, c0_v, c1_v, val_v, sem):
    wid = lax.axis_index("s") * NC + lax.axis_index("c")
    zeros = jnp.zeros((L,), jnp.float32)

    for r in range(BPW):
        b = wid * BPW + r

        cp0 = pltpu.async_copy(c01_hbm.at[0, b], c0_v, sem)
        cp1 = pltpu.async_copy(c01_hbm.at[1, b], c1_v, sem)
        cp2 = pltpu.async_copy(val_hbm.at[b], val_v, sem)

        @pl.loop(0, NUM_PIX, step=4 * L)
        def _(i):
            acc_v[pl.ds(i, L)] = zeros
            acc_v[pl.ds(i + L, L)] = zeros
            acc_v[pl.ds(i + 2 * L, L)] = zeros
            acc_v[pl.ds(i + 3 * L, L)] = zeros

        cp0.wait()
        cp1.wait()
        cp2.wait()

        def _points(i):
            c0 = c0_v[pl.ds(i, L)]
            c1 = c1_v[pl.ds(i, L)]
            v = val_v[pl.ds(i, L)]
            f0 = c0.astype(jnp.int32)
            f1 = c1.astype(jnp.int32)
            fr0 = c0 - f0.astype(jnp.float32)
            fr1 = c1 - f1.astype(jnp.float32)
            bv = fr0 * v
            av = v - bv
            p_fc = av * fr1
            p_ff = av - p_fc
            p_cc = bv * fr1
            p_cf = bv - p_cc
            i_ff = f0 * SIZE + f1
            plsc.addupdate_scatter(acc_v, [i_ff], p_ff)
            plsc.addupdate_scatter(acc_v, [i_ff + 1], p_fc)
            plsc.addupdate_scatter(acc_v, [i_ff + SIZE], p_cf)
            plsc.addupdate_scatter(acc_v, [i_ff + SIZE + 1], p_cc)

        @plsc.parallel_loop(0, N, step=L, unroll=2)
        def _(i):
            _points(i)

        pltpu.sync_copy(acc_v.at[pl.ds(0, NUM_PIX)], out_hbm.at[b])


def kernel(values, coord):
    c01 = jnp.maximum(jnp.transpose(coord, (2, 0, 1)), 0.0)
    mesh = plsc.VectorSubcoreMesh(core_axis_name="c", subcore_axis_name="s")
    cp = pltpu.CompilerParams()
    if "needs_layout_passes" in pltpu.CompilerParams.__dataclass_fields__:
        cp = dataclasses.replace(cp, needs_layout_passes=False)
    splat = pl.kernel(
        _splat_body,
        out_type=jax.ShapeDtypeStruct((B, NUM_PIX), jnp.float32),
        mesh=mesh,
        scratch_types=[
            pltpu.VMEM((NUM_PIX + 2 * SIZE,), jnp.float32),
            pltpu.VMEM((N,), jnp.float32),
            pltpu.VMEM((N,), jnp.float32),
            pltpu.VMEM((N,), jnp.float32),
            pltpu.SemaphoreType.DMA,
        ],
        compiler_params=cp,
    )
    img = splat(c01, values)
    return img.reshape(B, 1, SIZE, SIZE)
